# Initial kernel scaffold; baseline (speedup 1.0000x reference)
import jax
import jax.numpy as jnp
from jax.experimental import pallas as pl


def kernel(input_ids, age_ids, token_type_ids, word_emb, token_type_emb, age_emb, pos_emb, ln_gamma, ln_beta):
    raise NotImplementedError("write your pallas kernel here")


# speedup vs baseline: 4.8042x; 4.8042x over previous
import functools

import jax
import jax.numpy as jnp
from jax import lax
from jax.experimental import pallas as pl
from jax.experimental.pallas import tpu as pltpu
from jax.experimental.pallas import tpu_sc as plsc

_GATHER_WINDOW = 128
_TC_BLOCK_TOKENS = 1024


def _sc_gather(table, flat_ids):
    T = flat_ids.shape[0]
    H = table.shape[1]
    ids2 = flat_ids.reshape(1, T)
    mesh = plsc.VectorSubcoreMesh(core_axis_name="c", subcore_axis_name="s")

    @functools.partial(
        pl.kernel,
        out_type=jax.ShapeDtypeStruct((T, H), table.dtype),
        mesh=mesh,
    )
    def gather_kernel(x_hbm, i_hbm, o_hbm):
        def body(i_vmem, o_vmem):
            pltpu.sync_copy(x_hbm.at[i_vmem.at[0]], o_vmem)

        pltpu.emit_pipeline(
            body,
            grid=(T // _GATHER_WINDOW,),
            in_specs=[
                pl.BlockSpec((1, _GATHER_WINDOW), index_map=lambda i: (0, i))
            ],
            out_specs=[
                pl.BlockSpec((_GATHER_WINDOW, H), index_map=lambda i: (i, 0))
            ],
            core_axis_name=("c", "s"),
            dimension_semantics=(pltpu.PARALLEL,),
        )(i_hbm, o_hbm)

    return gather_kernel(table, ids2)


def _tc_fuse(gathered, age_ids, tt_ids, tt_emb, age_emb, pos_emb, gamma, beta,
             ln_eps):
    T, H = gathered.shape
    S = pos_emb.shape[0]
    BT = _TC_BLOCK_TOKENS
    NB = T // BT
    KB = BT // S
    AGES = age_emb.shape[0]

    age_r = age_ids.reshape(NB, 1, BT)
    tt_r = tt_ids.reshape(NB, 1, BT)
    gamma2 = gamma.reshape(1, H)
    beta2 = beta.reshape(1, H)

    def body(g_ref, age_ref, tt_ref, tte_ref, agee_ref, pos_ref, gam_ref,
             bet_ref, o_ref):
        g = g_ref[...]
        age = age_ref[0, 0, :]
        tt = tt_ref[0, 0, :].astype(jnp.float32)

        age_oh = (age[:, None]
                  == lax.broadcasted_iota(jnp.int32, (1, AGES), 1)
                  ).astype(jnp.float32)
        age_v = jnp.dot(age_oh, agee_ref[...],
                        preferred_element_type=jnp.float32)

        ttf = tt[:, None]
        tt_v = tte_ref[0:1, :] * (1.0 - ttf) + tte_ref[1:2, :] * ttf

        pos = jnp.broadcast_to(pos_ref[...][None], (KB, S, H)).reshape(BT, H)

        emb = g + age_v + tt_v + pos
        mean = jnp.mean(emb, axis=-1, keepdims=True)
        cent = emb - mean
        var = jnp.mean(cent * cent, axis=-1, keepdims=True)
        inv = lax.rsqrt(var + float(ln_eps))
        o_ref[...] = cent * inv * gam_ref[...] + bet_ref[...]

    return pl.pallas_call(
        body,
        grid=(NB,),
        in_specs=[
            pl.BlockSpec((BT, H), lambda i: (i, 0)),
            pl.BlockSpec((1, 1, BT), lambda i: (i, 0, 0)),
            pl.BlockSpec((1, 1, BT), lambda i: (i, 0, 0)),
            pl.BlockSpec(tt_emb.shape, lambda i: (0, 0)),
            pl.BlockSpec(age_emb.shape, lambda i: (0, 0)),
            pl.BlockSpec((S, H), lambda i: (0, 0)),
            pl.BlockSpec((1, H), lambda i: (0, 0)),
            pl.BlockSpec((1, H), lambda i: (0, 0)),
        ],
        out_specs=pl.BlockSpec((BT, H), lambda i: (i, 0)),
        out_shape=jax.ShapeDtypeStruct((T, H), jnp.float32),
    )(gathered, age_r, tt_r, tt_emb, age_emb, pos_emb, gamma2, beta2)


def kernel(input_ids, age_ids, token_type_ids, word_emb, token_type_emb,
           age_emb, pos_emb, ln_gamma, ln_beta):
    B, S = input_ids.shape
    H = word_emb.shape[1]
    T = B * S

    flat_ids = input_ids.reshape(T)
    gathered = _sc_gather(word_emb, flat_ids)
    out = _tc_fuse(gathered, age_ids.reshape(T), token_type_ids.reshape(T),
                   token_type_emb, age_emb, pos_emb, ln_gamma, ln_beta,
                   1e-12)
    return out.reshape(B, S, H)


# speedup vs baseline: 5.5516x; 1.1556x over previous
import functools

import jax
import jax.numpy as jnp
from jax import lax
from jax.experimental import pallas as pl
from jax.experimental.pallas import tpu as pltpu
from jax.experimental.pallas import tpu_sc as plsc

_GATHER_WINDOW = 128
_TC_BLOCK_TOKENS = 2048
_COMB_ROWS = 128


def _sc_gather(table, flat_ids):
    T = flat_ids.shape[0]
    H = table.shape[1]
    ids2 = flat_ids.reshape(1, T)
    mesh = plsc.VectorSubcoreMesh(core_axis_name="c", subcore_axis_name="s")

    @functools.partial(
        pl.kernel,
        out_type=jax.ShapeDtypeStruct((T, H), table.dtype),
        mesh=mesh,
    )
    def gather_kernel(x_hbm, i_hbm, o_hbm):
        def body(i_vmem, o_vmem):
            pltpu.sync_copy(x_hbm.at[i_vmem.at[0]], o_vmem)

        pltpu.emit_pipeline(
            body,
            grid=(T // _GATHER_WINDOW,),
            in_specs=[
                pl.BlockSpec((1, _GATHER_WINDOW), index_map=lambda i: (0, i))
            ],
            out_specs=[
                pl.BlockSpec((_GATHER_WINDOW, H), index_map=lambda i: (i, 0))
            ],
            core_axis_name=("c", "s"),
            dimension_semantics=(pltpu.PARALLEL,),
        )(i_hbm, o_hbm)

    return gather_kernel(table, ids2)


def _tc_fuse(gathered, age_ids, tt_ids, tt_emb, age_emb, pos_emb, gamma, beta,
             ln_eps):
    T, H = gathered.shape
    S = pos_emb.shape[0]
    BT = _TC_BLOCK_TOKENS
    NB = T // BT
    KB = BT // S
    AGES = age_emb.shape[0]

    comb = jnp.zeros((_COMB_ROWS, H), jnp.bfloat16)
    comb = comb.at[:AGES].set(age_emb.astype(jnp.bfloat16))
    comb = comb.at[AGES:AGES + tt_emb.shape[0]].set(
        tt_emb.astype(jnp.bfloat16))

    age_r = age_ids.reshape(NB, 1, BT)
    tt_r = tt_ids.reshape(NB, 1, BT)
    gamma2 = gamma.reshape(1, H)
    beta2 = beta.reshape(1, H)

    def body(g_ref, age_ref, tt_ref, comb_ref, pos_ref, gam_ref,
             bet_ref, o_ref):
        g = g_ref[...]
        age = age_ref[0, 0, :][:, None]
        tt = tt_ref[0, 0, :][:, None]

        col = lax.broadcasted_iota(jnp.int32, (1, _COMB_ROWS), 1)
        oh = ((age == col).astype(jnp.bfloat16)
              + (tt + AGES == col).astype(jnp.bfloat16))
        small_v = jnp.dot(oh, comb_ref[...],
                          preferred_element_type=jnp.float32)

        pos = jnp.broadcast_to(pos_ref[...][None], (KB, S, H)).reshape(BT, H)

        emb = g + small_v + pos
        mean = jnp.mean(emb, axis=-1, keepdims=True)
        cent = emb - mean
        var = jnp.mean(cent * cent, axis=-1, keepdims=True)
        inv = lax.rsqrt(var + float(ln_eps))
        o_ref[...] = cent * inv * gam_ref[...] + bet_ref[...]

    return pl.pallas_call(
        body,
        grid=(NB,),
        in_specs=[
            pl.BlockSpec((BT, H), lambda i: (i, 0)),
            pl.BlockSpec((1, 1, BT), lambda i: (i, 0, 0)),
            pl.BlockSpec((1, 1, BT), lambda i: (i, 0, 0)),
            pl.BlockSpec((_COMB_ROWS, H), lambda i: (0, 0)),
            pl.BlockSpec((S, H), lambda i: (0, 0)),
            pl.BlockSpec((1, H), lambda i: (0, 0)),
            pl.BlockSpec((1, H), lambda i: (0, 0)),
        ],
        out_specs=pl.BlockSpec((BT, H), lambda i: (i, 0)),
        out_shape=jax.ShapeDtypeStruct((T, H), jnp.float32),
        compiler_params=pltpu.CompilerParams(
            dimension_semantics=("arbitrary",)),
    )(gathered, age_r, tt_r, comb, pos_emb, gamma2, beta2)


def kernel(input_ids, age_ids, token_type_ids, word_emb, token_type_emb,
           age_emb, pos_emb, ln_gamma, ln_beta):
    B, S = input_ids.shape
    H = word_emb.shape[1]
    T = B * S

    flat_ids = input_ids.reshape(T)
    gathered = _sc_gather(word_emb, flat_ids)
    out = _tc_fuse(gathered, age_ids.reshape(T), token_type_ids.reshape(T),
                   token_type_emb, age_emb, pos_emb, ln_gamma, ln_beta,
                   1e-12)
    return out.reshape(B, S, H)


# speedup vs baseline: 6.1776x; 1.1128x over previous
import functools

import jax
import jax.numpy as jnp
from jax import lax
from jax.experimental import pallas as pl
from jax.experimental.pallas import tpu as pltpu
from jax.experimental.pallas import tpu_sc as plsc

_GATHER_WINDOW = 128
_TC_BLOCK_TOKENS = 2048
_COMB_ROWS = 128


def _sc_gather(table, flat_ids):
    T = flat_ids.shape[0]
    H = table.shape[1]
    ids2 = flat_ids.reshape(1, T)
    mesh = plsc.VectorSubcoreMesh(core_axis_name="c", subcore_axis_name="s")

    @functools.partial(
        pl.kernel,
        out_type=jax.ShapeDtypeStruct((T, H), table.dtype),
        mesh=mesh,
    )
    def gather_kernel(x_hbm, i_hbm, o_hbm):
        def body(i_vmem, o_vmem):
            pltpu.sync_copy(x_hbm.at[i_vmem.at[0]], o_vmem)

        pltpu.emit_pipeline(
            body,
            grid=(T // _GATHER_WINDOW,),
            in_specs=[
                pl.BlockSpec((1, _GATHER_WINDOW), index_map=lambda i: (0, i))
            ],
            out_specs=[
                pl.BlockSpec((_GATHER_WINDOW, H), index_map=lambda i: (i, 0))
            ],
            core_axis_name=("c", "s"),
            dimension_semantics=(pltpu.PARALLEL,),
        )(i_hbm, o_hbm)

    return gather_kernel(table, ids2)


def _tc_fuse_chunk(acc, gathered_c, age_r, tt_r, comb, pos_emb, gamma2, beta2,
                   chunk, T, ln_eps):
    Tc, H = gathered_c.shape
    S = pos_emb.shape[0]
    BT = _TC_BLOCK_TOKENS
    NBc = Tc // BT
    KB = BT // S
    AGES = 110
    base = chunk * NBc

    def body(*refs):
        if acc is None:
            g_ref, age_ref, tt_ref, comb_ref, pos_ref, gam_ref, bet_ref, \
                o_ref = refs
        else:
            _, g_ref, age_ref, tt_ref, comb_ref, pos_ref, gam_ref, bet_ref, \
                o_ref = refs
        g = g_ref[...]
        age = age_ref[0, 0, :][:, None]
        tt = tt_ref[0, 0, :][:, None]

        col = lax.broadcasted_iota(jnp.int32, (1, _COMB_ROWS), 1)
        oh = ((age == col).astype(jnp.bfloat16)
              + (tt + AGES == col).astype(jnp.bfloat16))
        small_v = jnp.dot(oh, comb_ref[...],
                          preferred_element_type=jnp.float32)

        pos = jnp.broadcast_to(pos_ref[...][None], (KB, S, H)).reshape(BT, H)

        emb = g + small_v + pos
        mean = jnp.mean(emb, axis=-1, keepdims=True)
        cent = emb - mean
        var = jnp.mean(cent * cent, axis=-1, keepdims=True)
        inv = lax.rsqrt(var + float(ln_eps))
        o_ref[...] = cent * inv * gam_ref[...] + bet_ref[...]

    in_specs = [
        pl.BlockSpec((BT, H), lambda i: (i, 0)),
        pl.BlockSpec((1, 1, BT), lambda i: (i, 0, 0)),
        pl.BlockSpec((1, 1, BT), lambda i: (i, 0, 0)),
        pl.BlockSpec((_COMB_ROWS, H), lambda i: (0, 0)),
        pl.BlockSpec((S, H), lambda i: (0, 0)),
        pl.BlockSpec((1, H), lambda i: (0, 0)),
        pl.BlockSpec((1, H), lambda i: (0, 0)),
    ]
    args = [gathered_c, age_r, tt_r, comb, pos_emb, gamma2, beta2]
    aliases = {}
    if acc is not None:
        in_specs = [pl.BlockSpec(memory_space=pl.ANY)] + in_specs
        args = [acc] + args
        aliases = {0: 0}

    return pl.pallas_call(
        body,
        grid=(NBc,),
        in_specs=in_specs,
        out_specs=pl.BlockSpec((BT, H), lambda i: (i + base, 0)),
        out_shape=jax.ShapeDtypeStruct((T, H), jnp.float32),
        input_output_aliases=aliases,
        compiler_params=pltpu.CompilerParams(
            dimension_semantics=("arbitrary",)),
    )(*args)


_NUM_CHUNKS = 8


def kernel(input_ids, age_ids, token_type_ids, word_emb, token_type_emb,
           age_emb, pos_emb, ln_gamma, ln_beta):
    B, S = input_ids.shape
    H = word_emb.shape[1]
    T = B * S
    C = _NUM_CHUNKS
    Tc = T // C
    BT = _TC_BLOCK_TOKENS
    NBc = Tc // BT
    AGES = age_emb.shape[0]

    comb = jnp.zeros((_COMB_ROWS, H), jnp.bfloat16)
    comb = comb.at[:AGES].set(age_emb.astype(jnp.bfloat16))
    comb = comb.at[AGES:AGES + token_type_emb.shape[0]].set(
        token_type_emb.astype(jnp.bfloat16))

    flat_ids = input_ids.reshape(C, Tc)
    age_r = age_ids.reshape(C, NBc, 1, BT)
    tt_r = token_type_ids.reshape(C, NBc, 1, BT)
    gamma2 = ln_gamma.reshape(1, H)
    beta2 = ln_beta.reshape(1, H)

    gathered = [_sc_gather(word_emb, flat_ids[c]) for c in range(C)]
    acc = None
    for c in range(C):
        acc = _tc_fuse_chunk(acc, gathered[c], age_r[c], tt_r[c], comb,
                             pos_emb, gamma2, beta2, c, T, 1e-12)
    return acc.reshape(B, S, H)
